# Initial kernel scaffold; baseline (speedup 1.0000x reference)
import jax
import jax.numpy as jnp
from jax.experimental import pallas as pl


def kernel(input_ids, token_type_ids, word_table, pos_table, type_table, gamma, beta, W, b):
    raise NotImplementedError("write your pallas kernel here")


# speedup vs baseline: 1.5137x; 1.5137x over previous
import functools

import jax
import jax.numpy as jnp
from jax import lax
from jax.experimental import pallas as pl
from jax.experimental.pallas import tpu as pltpu
from jax.experimental.pallas import tpu_sc as plsc

EPS = 1e-12


def _make_sc_gather(n_rows, emb, vocab):
    info = plsc.get_sparse_core_info()
    nc, ns = info.num_cores, info.num_subcores
    nw = nc * ns
    rows_per_w = n_rows // nw
    chunk = 128
    n_chunks = rows_per_w // chunk
    mesh = plsc.VectorSubcoreMesh(core_axis_name="c", subcore_axis_name="s")

    @functools.partial(
        pl.kernel,
        mesh=mesh,
        out_type=jax.ShapeDtypeStruct((n_rows, emb), jnp.float32),
        scratch_types=[
            pltpu.VMEM((n_chunks, chunk), jnp.int32),
            pltpu.VMEM((rows_per_w, emb), jnp.float32),
            pltpu.SemaphoreType.DMA,
        ],
    )
    def gather_kernel(table_hbm, idx_hbm, out_hbm, idx_v, rows_v, sem):
        wid = lax.axis_index("s") * nc + lax.axis_index("c")
        pltpu.sync_copy(idx_hbm.at[pl.ds(wid * n_chunks, n_chunks)], idx_v)
        copies = []
        for j in range(n_chunks):
            copies.append(
                pltpu.async_copy(
                    table_hbm.at[idx_v.at[j]],
                    rows_v.at[pl.ds(j * chunk, chunk)],
                    sem,
                )
            )
        for c in copies:
            c.wait()
        pltpu.sync_copy(rows_v, out_hbm.at[pl.ds(wid * rows_per_w, rows_per_w)])

    return gather_kernel


def _tc_body(emb_ref, pos_ref, tt_ref, type_ref, gamma_ref, beta_ref,
             w_ref, b_ref, out_ref):
    x = emb_ref[...] + pos_ref[...]
    t0 = type_ref[0:1, :]
    t1 = type_ref[1:2, :]
    x = x + t0 + tt_ref[...] * (t1 - t0)
    mean = jnp.mean(x, axis=1, keepdims=True)
    xc = x - mean
    var = jnp.mean(xc * xc, axis=1, keepdims=True)
    y = xc * lax.rsqrt(var + EPS) * gamma_ref[...] + beta_ref[...]
    out_ref[...] = (
        jnp.dot(y, w_ref[...], preferred_element_type=jnp.float32) + b_ref[...]
    )


def kernel(input_ids, token_type_ids, word_table, pos_table, type_table,
           gamma, beta, W, b):
    B, S = input_ids.shape
    n = B * S
    emb = word_table.shape[1]
    hid = W.shape[1]
    vocab = word_table.shape[0]

    ids_flat = input_ids.reshape(n).astype(jnp.int32)
    tt_col = token_type_ids.reshape(n, 1).astype(jnp.float32)

    ids_2d = ids_flat.reshape(n // 128, 128)
    gathered = _make_sc_gather(n, emb, vocab)(word_table, ids_2d)

    blk = 512
    grid = (n // blk,)
    n_pos_blocks = S // blk
    out_flat = pl.pallas_call(
        _tc_body,
        grid=grid,
        in_specs=[
            pl.BlockSpec((blk, emb), lambda i: (i, 0)),
            pl.BlockSpec((blk, emb), lambda i: (i % n_pos_blocks, 0)),
            pl.BlockSpec((blk, 1), lambda i: (i, 0)),
            pl.BlockSpec((2, emb), lambda i: (0, 0)),
            pl.BlockSpec((1, emb), lambda i: (0, 0)),
            pl.BlockSpec((1, emb), lambda i: (0, 0)),
            pl.BlockSpec((emb, hid), lambda i: (0, 0)),
            pl.BlockSpec((1, hid), lambda i: (0, 0)),
        ],
        out_specs=pl.BlockSpec((blk, hid), lambda i: (i, 0)),
        out_shape=jax.ShapeDtypeStruct((n, hid), jnp.float32),
    )(
        gathered,
        pos_table,
        tt_col,
        type_table,
        gamma.reshape(1, emb),
        beta.reshape(1, emb),
        W,
        b.reshape(1, hid),
    )
    return out_flat.reshape(B, S, hid)


# speedup vs baseline: 1.5282x; 1.0096x over previous
import functools

import jax
import jax.numpy as jnp
from jax import lax
from jax.experimental import pallas as pl
from jax.experimental.pallas import tpu as pltpu
from jax.experimental.pallas import tpu_sc as plsc

EPS = 1e-12


def _make_sc_gather(n_rows, emb, vocab):
    info = plsc.get_sparse_core_info()
    nc, ns = info.num_cores, info.num_subcores
    nw = nc * ns
    rows_per_w = n_rows // nw
    chunk = 128
    n_chunks = rows_per_w // chunk
    mesh = plsc.VectorSubcoreMesh(core_axis_name="c", subcore_axis_name="s")

    @functools.partial(
        pl.kernel,
        mesh=mesh,
        out_type=jax.ShapeDtypeStruct((n_rows, emb), jnp.float32),
        scratch_types=[
            pltpu.VMEM((n_chunks, chunk), jnp.int32),
            pltpu.VMEM((rows_per_w, emb), jnp.float32),
            pltpu.SemaphoreType.DMA,
        ],
    )
    def gather_kernel(table_hbm, idx_hbm, out_hbm, idx_v, rows_v, sem):
        wid = lax.axis_index("s") * nc + lax.axis_index("c")
        pltpu.sync_copy(idx_hbm.at[pl.ds(wid * n_chunks, n_chunks)], idx_v)
        copies = []
        for j in range(n_chunks):
            copies.append(
                pltpu.async_copy(
                    table_hbm.at[idx_v.at[j]],
                    rows_v.at[pl.ds(j * chunk, chunk)],
                    sem,
                )
            )
        for c in copies:
            c.wait()
        pltpu.sync_copy(rows_v, out_hbm.at[pl.ds(wid * rows_per_w, rows_per_w)])

    return gather_kernel


def _tc_body(emb_ref, pos_ref, tt_ref, type_ref, gamma_ref, beta_ref,
             w_ref, b_ref, out_ref):
    x = emb_ref[...] + pos_ref[...]
    t0 = type_ref[0:1, :]
    t1 = type_ref[1:2, :]
    x = x + t0 + tt_ref[...] * (t1 - t0)
    mean = jnp.mean(x, axis=1, keepdims=True)
    xc = x - mean
    var = jnp.mean(xc * xc, axis=1, keepdims=True)
    y = xc * lax.rsqrt(var + EPS) * gamma_ref[...] + beta_ref[...]
    out_ref[...] = (
        jnp.dot(y.astype(jnp.bfloat16), w_ref[...],
                preferred_element_type=jnp.float32) + b_ref[...]
    )


def kernel(input_ids, token_type_ids, word_table, pos_table, type_table,
           gamma, beta, W, b):
    B, S = input_ids.shape
    n = B * S
    emb = word_table.shape[1]
    hid = W.shape[1]
    vocab = word_table.shape[0]

    ids_flat = input_ids.reshape(n).astype(jnp.int32)
    tt_col = token_type_ids.reshape(n, 1).astype(jnp.float32)

    ids_2d = ids_flat.reshape(n // 128, 128)
    gathered = _make_sc_gather(n, emb, vocab)(word_table, ids_2d)

    blk = 512
    n_pos_blocks = S // blk
    grid = (n_pos_blocks, B)
    row_ix = lambda p, bb: (bb * n_pos_blocks + p, 0)
    out_flat = pl.pallas_call(
        _tc_body,
        grid=grid,
        in_specs=[
            pl.BlockSpec((blk, emb), row_ix),
            pl.BlockSpec((blk, emb), lambda p, bb: (p, 0)),
            pl.BlockSpec((blk, 1), row_ix),
            pl.BlockSpec((2, emb), lambda p, bb: (0, 0)),
            pl.BlockSpec((1, emb), lambda p, bb: (0, 0)),
            pl.BlockSpec((1, emb), lambda p, bb: (0, 0)),
            pl.BlockSpec((emb, hid), lambda p, bb: (0, 0)),
            pl.BlockSpec((1, hid), lambda p, bb: (0, 0)),
        ],
        out_specs=pl.BlockSpec((blk, hid), row_ix),
        out_shape=jax.ShapeDtypeStruct((n, hid), jnp.float32),
    )(
        gathered,
        pos_table,
        tt_col,
        type_table,
        gamma.reshape(1, emb),
        beta.reshape(1, emb),
        W.astype(jnp.bfloat16),
        b.reshape(1, hid),
    )
    return out_flat.reshape(B, S, hid)


# speedup vs baseline: 1.5445x; 1.0106x over previous
import functools

import jax
import jax.numpy as jnp
from jax import lax
from jax.experimental import pallas as pl
from jax.experimental.pallas import tpu as pltpu
from jax.experimental.pallas import tpu_sc as plsc

EPS = 1e-12


def _make_sc_gather(n_rows, emb, vocab):
    info = plsc.get_sparse_core_info()
    nc, ns = info.num_cores, info.num_subcores
    nw = nc * ns
    rows_per_w = n_rows // nw
    chunk = 128
    n_chunks = rows_per_w // chunk
    mesh = plsc.VectorSubcoreMesh(core_axis_name="c", subcore_axis_name="s")

    @functools.partial(
        pl.kernel,
        mesh=mesh,
        out_type=jax.ShapeDtypeStruct((n_rows, emb), jnp.float32),
        scratch_types=[
            pltpu.VMEM((n_chunks, chunk), jnp.int32),
            pltpu.VMEM((rows_per_w, emb), jnp.float32),
            pltpu.SemaphoreType.DMA,
        ],
    )
    def gather_kernel(table_hbm, idx_hbm, out_hbm, idx_v, rows_v, sem):
        wid = lax.axis_index("s") * nc + lax.axis_index("c")
        pltpu.sync_copy(idx_hbm.at[pl.ds(wid * n_chunks, n_chunks)], idx_v)
        copies = []
        for j in range(n_chunks):
            copies.append(
                pltpu.async_copy(
                    table_hbm.at[idx_v.at[j]],
                    rows_v.at[pl.ds(j * chunk, chunk)],
                    sem,
                )
            )
        for c in copies:
            c.wait()
        pltpu.sync_copy(rows_v, out_hbm.at[pl.ds(wid * rows_per_w, rows_per_w)])

    return gather_kernel


def _tc_body(emb_ref, pos_ref, tt_ref, tdiff_ref, w_ref, b_ref, out_ref):
    x = emb_ref[...] + pos_ref[...] + tt_ref[...] * tdiff_ref[...]
    mean = jnp.mean(x, axis=1, keepdims=True)
    msq = jnp.mean(x * x, axis=1, keepdims=True)
    var = msq - mean * mean
    xc = x - mean
    y = xc * lax.rsqrt(var + EPS)
    out_ref[...] = (
        jnp.dot(y.astype(jnp.bfloat16), w_ref[...],
                preferred_element_type=jnp.float32) + b_ref[...]
    )


def kernel(input_ids, token_type_ids, word_table, pos_table, type_table,
           gamma, beta, W, b):
    B, S = input_ids.shape
    n = B * S
    emb = word_table.shape[1]
    hid = W.shape[1]
    vocab = word_table.shape[0]

    ids_flat = input_ids.reshape(n).astype(jnp.int32)
    tt_col = token_type_ids.reshape(n, 1).astype(jnp.float32)

    ids_2d = ids_flat.reshape(n // 128, 128)
    gathered = _make_sc_gather(n, emb, vocab)(word_table, ids_2d)

    w_eff = (gamma[:, None] * W).astype(jnp.bfloat16)
    b_eff = (beta @ W + b).reshape(1, hid)
    pos_eff = pos_table + type_table[0][None, :]
    tdiff = (type_table[1] - type_table[0]).reshape(1, emb)

    blk = 512
    n_pos_blocks = S // blk
    grid = (n_pos_blocks, B)
    row_ix = lambda p, bb: (bb * n_pos_blocks + p, 0)
    out_flat = pl.pallas_call(
        _tc_body,
        grid=grid,
        in_specs=[
            pl.BlockSpec((blk, emb), row_ix),
            pl.BlockSpec((blk, emb), lambda p, bb: (p, 0)),
            pl.BlockSpec((blk, 1), row_ix),
            pl.BlockSpec((1, emb), lambda p, bb: (0, 0)),
            pl.BlockSpec((emb, hid), lambda p, bb: (0, 0)),
            pl.BlockSpec((1, hid), lambda p, bb: (0, 0)),
        ],
        out_specs=pl.BlockSpec((blk, hid), row_ix),
        out_shape=jax.ShapeDtypeStruct((n, hid), jnp.float32),
    )(
        gathered,
        pos_eff,
        tt_col,
        tdiff,
        w_eff,
        b_eff,
    )
    return out_flat.reshape(B, S, hid)


# speedup vs baseline: 1.5461x; 1.0010x over previous
import functools

import jax
import jax.numpy as jnp
from jax import lax
from jax.experimental import pallas as pl
from jax.experimental.pallas import tpu as pltpu
from jax.experimental.pallas import tpu_sc as plsc

EPS = 1e-12


def _make_sc_gather(n_rows, emb, vocab):
    info = plsc.get_sparse_core_info()
    nc, ns = info.num_cores, info.num_subcores
    nw = nc * ns
    rows_per_w = n_rows // nw
    chunk = 128
    n_chunks = rows_per_w // chunk
    mesh = plsc.VectorSubcoreMesh(core_axis_name="c", subcore_axis_name="s")

    @functools.partial(
        pl.kernel,
        mesh=mesh,
        out_type=jax.ShapeDtypeStruct((n_rows, emb), jnp.float32),
        scratch_types=[
            pltpu.VMEM((n_chunks, chunk), jnp.int32),
            pltpu.VMEM((rows_per_w, emb), jnp.float32),
            pltpu.SemaphoreType.DMA,
        ],
    )
    def gather_kernel(table_hbm, idx_hbm, out_hbm, idx_v, rows_v, sem):
        wid = lax.axis_index("s") * nc + lax.axis_index("c")
        pltpu.sync_copy(idx_hbm.at[pl.ds(wid * n_chunks, n_chunks)], idx_v)
        copies = []
        for j in range(n_chunks):
            copies.append(
                pltpu.async_copy(
                    table_hbm.at[idx_v.at[j]],
                    rows_v.at[pl.ds(j * chunk, chunk)],
                    sem,
                )
            )
        for c in copies:
            c.wait()
        pltpu.sync_copy(rows_v, out_hbm.at[pl.ds(wid * rows_per_w, rows_per_w)])

    return gather_kernel


def _tc_body(emb_ref, pos_ref, tt_ref, tdiff_ref, w_ref, b_ref, out_ref):
    x = emb_ref[...] + pos_ref[...] + tt_ref[...] * tdiff_ref[...]
    mean = jnp.mean(x, axis=1, keepdims=True)
    msq = jnp.mean(x * x, axis=1, keepdims=True)
    var = msq - mean * mean
    xc = x - mean
    y = xc * lax.rsqrt(var + EPS)
    out_ref[...] = (
        jnp.dot(y.astype(jnp.bfloat16), w_ref[...],
                preferred_element_type=jnp.float32) + b_ref[...]
    )


def kernel(input_ids, token_type_ids, word_table, pos_table, type_table,
           gamma, beta, W, b):
    B, S = input_ids.shape
    n = B * S
    emb = word_table.shape[1]
    hid = W.shape[1]
    vocab = word_table.shape[0]

    ids_flat = input_ids.reshape(n).astype(jnp.int32)
    tt_col = token_type_ids.reshape(n, 1).astype(jnp.float32)

    ids_2d = ids_flat.reshape(n // 128, 128)
    gathered = _make_sc_gather(n, emb, vocab)(word_table, ids_2d)

    w_eff = (gamma[:, None] * W).astype(jnp.bfloat16)
    b_eff = (beta @ W + b).reshape(1, hid)
    pos_eff = pos_table + type_table[0][None, :]
    tdiff = (type_table[1] - type_table[0]).reshape(1, emb)

    blk = 512
    n_pos_blocks = S // blk
    grid = (n_pos_blocks, B)
    row_ix = lambda p, bb: (bb * n_pos_blocks + p, 0)
    out_flat = pl.pallas_call(
        _tc_body,
        grid=grid,
        in_specs=[
            pl.BlockSpec((blk, emb), row_ix),
            pl.BlockSpec((blk, emb), lambda p, bb: (p, 0)),
            pl.BlockSpec((blk, 1), row_ix),
            pl.BlockSpec((1, emb), lambda p, bb: (0, 0)),
            pl.BlockSpec((emb, hid), lambda p, bb: (0, 0)),
            pl.BlockSpec((1, hid), lambda p, bb: (0, 0)),
        ],
        out_specs=pl.BlockSpec((blk, hid), row_ix),
        out_shape=jax.ShapeDtypeStruct((n, hid), jnp.float32),
        compiler_params=pltpu.CompilerParams(
            dimension_semantics=("parallel", "parallel"),
        ),
    )(
        gathered,
        pos_eff,
        tt_col,
        tdiff,
        w_eff,
        b_eff,
    )
    return out_flat.reshape(B, S, hid)
